# baseline (speedup 1.0000x reference)
import functools

import jax
import jax.numpy as jnp
from jax import lax
from jax.experimental import pallas as pl
from jax.experimental.pallas import tpu as pltpu
from jax.experimental.pallas import tpu_sc as plsc

VOCAB = 1000000
D = 64
BATCH = 4096
HIST = 200

NC, NS = 2, 16
NW = NC * NS
B_TOT = BATCH * HIST
BPW = B_TOT // NW
CHUNK = 128
NCHUNK = BPW // CHUNK
NBUF = 4


def _body(idx_hbm, table_hbm, out_hbm, idx_v, rows0, rows1, rows2, rows3,
          g0, g1, g2, g3):
  rows = (rows0, rows1, rows2, rows3)
  gsem = (g0, g1, g2, g3)
  c = lax.axis_index("c")
  s = lax.axis_index("s")
  wid = s * NC + c

  pltpu.sync_copy(idx_hbm.at[wid], idx_v)

  base = wid * BPW

  def start_gather(j, b):
    pltpu.make_async_copy(
        table_hbm.at[idx_v.at[j]], rows[b], gsem[b]).start()

  def wait_gather(b):
    pltpu.make_async_copy(
        table_hbm.at[idx_v.at[0]], rows[b], gsem[b]).wait()

  for b in range(NBUF):
    start_gather(b, b)

  @pl.loop(0, NCHUNK - NBUF, step=NBUF)
  def _(jj):
    for b in range(NBUF):
      j = jj + b
      wait_gather(b)
      pltpu.sync_copy(rows[b], out_hbm.at[pl.ds(base + j * CHUNK, CHUNK)])
      start_gather(j + NBUF, b)

  for b in range(NBUF):
    j = NCHUNK - NBUF + b
    wait_gather(b)
    pltpu.sync_copy(rows[b], out_hbm.at[pl.ds(base + j * CHUNK, CHUNK)])


@jax.jit
def _embed(x_flat, table):
  mesh = plsc.VectorSubcoreMesh(
      core_axis_name="c", subcore_axis_name="s", num_cores=NC,
      num_subcores=NS)
  run = functools.partial(
      pl.kernel,
      out_type=jax.ShapeDtypeStruct((B_TOT, D), jnp.float32),
      mesh=mesh,
      compiler_params=pltpu.CompilerParams(use_tc_tiling_on_sc=False),
      scratch_types=(
          [pltpu.VMEM((NCHUNK, CHUNK), jnp.int32)]
          + [pltpu.VMEM((CHUNK, D), jnp.float32) for _ in range(NBUF)]
          + [pltpu.SemaphoreType.DMA for _ in range(NBUF)]
      ),
  )(_body)
  return run(x_flat, table)


def kernel(x, table):
  x_flat = x.reshape(NW, NCHUNK, CHUNK).astype(jnp.int32)
  out = _embed(x_flat, table)
  return out.reshape(BATCH, HIST, D)
